# Initial kernel scaffold; baseline (speedup 1.0000x reference)
import functools

import jax
import jax.numpy as jnp
from jax import lax
from jax.experimental import pallas as pl
from jax.experimental.pallas import tpu as pltpu
from jax.experimental.pallas import tpu_sc as plsc

NC = 2
NS = 16
NW = NC * NS
CHUNK = 128
B_TOTAL = 4096 * 50
B_PER_W = B_TOTAL // NW
N_CHUNKS = B_PER_W // CHUNK
EMBED = 64

_mesh = plsc.VectorSubcoreMesh(
    core_axis_name="c", subcore_axis_name="s", num_cores=NC, num_subcores=NS
)


@functools.partial(
    pl.kernel,
    out_type=jax.ShapeDtypeStruct((B_TOTAL, EMBED), jnp.float32),
    mesh=_mesh,
    scratch_types=[
        pltpu.VMEM((N_CHUNKS, CHUNK), jnp.int32),
        pltpu.VMEM((CHUNK, EMBED), jnp.float32),
        pltpu.SemaphoreType.DMA,
    ],
)
def _embed_gather(idx_hbm, table_hbm, out_hbm, idx_v, rows_v, sem):
    wid = lax.axis_index("s") * NC + lax.axis_index("c")
    base = wid * B_PER_W
    pltpu.sync_copy(idx_hbm.at[wid], idx_v)

    @pl.loop(0, N_CHUNKS)
    def _step(j):
        pltpu.async_copy(table_hbm.at[idx_v.at[j]], rows_v, sem).wait()
        pltpu.sync_copy(rows_v, out_hbm.at[pl.ds(base + j * CHUNK, CHUNK)])


def kernel(input, weight):
    idx = input.reshape(NW, N_CHUNKS, CHUNK)
    out = _embed_gather(idx, weight)
    return out.reshape(input.shape + (EMBED,))


# speedup vs baseline: 4.0849x; 4.0849x over previous
import functools

import jax
import jax.numpy as jnp
from jax import lax
from jax.experimental import pallas as pl
from jax.experimental.pallas import tpu as pltpu
from jax.experimental.pallas import tpu_sc as plsc

NC = 2
NS = 16
NW = NC * NS
CHUNK = 128
B_TOTAL = 4096 * 50
B_PER_W = B_TOTAL // NW
N_CHUNKS = B_PER_W // CHUNK
EMBED = 64

_mesh = plsc.VectorSubcoreMesh(
    core_axis_name="c", subcore_axis_name="s", num_cores=NC, num_subcores=NS
)


@functools.partial(
    pl.kernel,
    out_type=jax.ShapeDtypeStruct((B_TOTAL, EMBED), jnp.float32),
    mesh=_mesh,
    scratch_types=[
        pltpu.VMEM((N_CHUNKS, CHUNK), jnp.int32),
        pltpu.VMEM((CHUNK, EMBED), jnp.float32),
        pltpu.SemaphoreType.DMA,
    ],
    compiler_params=pltpu.CompilerParams(use_tc_tiling_on_sc=False),
)
def _embed_gather(idx_hbm, table_hbm, out_hbm, idx_v, rows_v, sem):
    wid = lax.axis_index("s") * NC + lax.axis_index("c")
    base = wid * B_PER_W
    pltpu.sync_copy(idx_hbm.at[wid], idx_v)

    @pl.loop(0, N_CHUNKS)
    def _step(j):
        pltpu.async_copy(table_hbm.at[idx_v.at[j]], rows_v, sem).wait()
        pltpu.sync_copy(rows_v, out_hbm.at[pl.ds(base + j * CHUNK, CHUNK)])


def kernel(input, weight):
    idx = input.reshape(NW, N_CHUNKS, CHUNK)
    out = _embed_gather(idx, weight)
    return out.reshape(input.shape + (EMBED,))


# speedup vs baseline: 4.6717x; 1.1436x over previous
import functools

import jax
import jax.numpy as jnp
from jax import lax
from jax.experimental import pallas as pl
from jax.experimental.pallas import tpu as pltpu
from jax.experimental.pallas import tpu_sc as plsc

NC = 2
NS = 16
NW = NC * NS
BATCH = 4096
HIST = 50
EMBED = 64
NB = BATCH // NW
K = 4
G = NB // K
NBUF = 4

_mesh = plsc.VectorSubcoreMesh(
    core_axis_name="c", subcore_axis_name="s", num_cores=NC, num_subcores=NS
)


@functools.partial(
    pl.kernel,
    out_type=jax.ShapeDtypeStruct((BATCH, HIST, EMBED), jnp.float32),
    mesh=_mesh,
    scratch_types=[
        pltpu.VMEM((NB, HIST), jnp.int32),
    ]
    + [pltpu.VMEM((K, HIST, EMBED), jnp.float32) for _ in range(NBUF)]
    + [pltpu.SemaphoreType.DMA for _ in range(2 * NBUF)],
    compiler_params=pltpu.CompilerParams(use_tc_tiling_on_sc=False),
)
def _embed_gather(idx_hbm, table_hbm, out_hbm, idx_v, *bufs_and_sems):
    bufs = bufs_and_sems[:NBUF]
    gsem = bufs_and_sems[NBUF : 2 * NBUF]
    ssem = bufs_and_sems[2 * NBUF : 3 * NBUF]

    wid = lax.axis_index("s") * NC + lax.axis_index("c")
    b0 = wid * NB
    pltpu.sync_copy(idx_hbm.at[pl.ds(b0, NB)], idx_v)

    def fire_gather(g, b):
        for j in range(K):
            pltpu.async_copy(
                table_hbm.at[idx_v.at[g * K + j]], bufs[b].at[j], gsem[b]
            )

    def wait_gather(g, b):
        for j in range(K):
            pltpu.make_async_copy(
                table_hbm.at[idx_v.at[g * K + j]], bufs[b].at[j], gsem[b]
            ).wait()

    for b in range(NBUF):
        fire_gather(b, b)

    @pl.loop(0, G, step=NBUF)
    def _outer(g0):
        for b in range(NBUF):
            g = g0 + b
            out_slc = out_hbm.at[pl.ds(b0 + g * K, K)]
            wait_gather(g, b)
            pltpu.async_copy(bufs[b], out_slc, ssem[b])

            @pl.when(g + NBUF < G)
            def _():
                pltpu.make_async_copy(bufs[b], out_slc, ssem[b]).wait()
                fire_gather(g + NBUF, b)

    for b in range(NBUF):
        g = G - NBUF + b
        pltpu.make_async_copy(
            bufs[b], out_hbm.at[pl.ds(b0 + g * K, K)], ssem[b]
        ).wait()


def kernel(input, weight):
    return _embed_gather(input, weight)
